# baseline (speedup 1.0000x reference)
import jax
import jax.numpy as jnp
from jax.experimental import pallas as pl

N = 50000
G = 128
BN = 1000


def _sage_linear(mean, h, Wl, Wr, b):
    def body(m_ref, h_ref, wl_ref, wr_ref, b_ref, o_ref):
        acc = (m_ref[...] @ wl_ref[...] + h_ref[...] @ wr_ref[...]
               + b_ref[...][None, :])
        o_ref[...] = jnp.maximum(acc, 0.0)

    n = mean.shape[0]
    d = Wl.shape[0]
    hdim = Wl.shape[1]
    grid = (n // BN,)
    return pl.pallas_call(
        body,
        grid=grid,
        in_specs=[
            pl.BlockSpec((BN, d), lambda i: (i, 0)),
            pl.BlockSpec((BN, d), lambda i: (i, 0)),
            pl.BlockSpec((d, hdim), lambda i: (0, 0)),
            pl.BlockSpec((d, hdim), lambda i: (0, 0)),
            pl.BlockSpec((hdim,), lambda i: (0,)),
        ],
        out_specs=pl.BlockSpec((BN, hdim), lambda i: (i, 0)),
        out_shape=jax.ShapeDtypeStruct((n, hdim), jnp.float32),
    )(mean, h, Wl, Wr, b)


def kernel(x, edge_index, batch, emb, W1l, W1r, b1, W2l, W2r, b2, Wlin, blin):
    src = edge_index[0]
    dst = edge_index[1]
    h = jnp.take(emb, x, axis=0)
    cnt = jax.ops.segment_sum(jnp.ones((src.shape[0], 1), jnp.float32), dst,
                              num_segments=N)
    denom = jnp.maximum(cnt, 1.0)

    msg = jnp.take(h, src, axis=0)
    mean1 = jax.ops.segment_sum(msg, dst, num_segments=N) / denom
    h1 = _sage_linear(mean1, h, W1l, W1r, b1)

    msg2 = jnp.take(h1, src, axis=0)
    mean2 = jax.ops.segment_sum(msg2, dst, num_segments=N) / denom
    h2 = _sage_linear(mean2, h1, W2l, W2r, b2)

    sums = jax.ops.segment_sum(h2, batch, num_segments=G)
    cnts = jax.ops.segment_sum(jnp.ones((N, 1), jnp.float32), batch,
                               num_segments=G)
    pooled = sums / jnp.maximum(cnts, 1.0)
    return pooled @ Wlin + blin


# speedup vs baseline: 2.7173x; 2.7173x over previous
import jax
import jax.numpy as jnp
from jax import lax
from jax.experimental import pallas as pl
from jax.experimental.pallas import tpu as pltpu
from jax.experimental.pallas import tpu_sc as plsc

N = 50000
E = 800000
G = 128
V = 64
D = 64
H = 64

NC = 2
NS = 16
NPH = 2
HALF = N // NC
QROWS = HALF // NPH

CH = 2048
SUB = 128
NSUB = CH // SUB
NCHUNK = 25
EPT = NCHUNK * CH
EPAD = EPT * NS

HWQ = 819200
HSPANQ = HWQ // NS
HZQ = HSPANQ // 4
HDUM = QROWS * V

WQ = H // 2
RQR = 25600
RSPANQ = RQR // NS
RZQ = RSPANQ // 4
RDUM = HALF

BN = 1000
NB = N // BN


def _mesh():
    return plsc.VectorSubcoreMesh(core_axis_name="c", subcore_axis_name="s")


def _sc_hist(srcp, dstp, x):
    def body(src_h, dst_h, x_h, out_h, x_v, sv, dv, idx2d, ones_v,
             bounce, hist_sh):
        c = lax.axis_index("c")
        s = lax.axis_index("s")

        for i in range(SUB // 16):
            ones_v[pl.ds(i * 16, 16)] = jnp.ones((16,), jnp.float32)
        pltpu.sync_copy(x_h, x_v)

        zv = jnp.zeros((16,), jnp.float32)
        ebase = s * EPT
        base = s * HSPANQ

        for q in range(NPH):
            lo = c * HALF + q * QROWS

            def zi(i, _):
                bounce[pl.ds(i * 16, 16)] = zv
                return 0

            lax.fori_loop(0, HZQ // 16, zi, 0)
            for i in range(4):
                pltpu.sync_copy(bounce,
                                hist_sh.at[pl.ds(base + i * HZQ, HZQ)])
            plsc.subcore_barrier()

            def chunk(ci, _):
                off = ebase + ci * CH
                pltpu.sync_copy(src_h.at[pl.ds(off, CH)], sv)
                pltpu.sync_copy(dst_h.at[pl.ds(off, CH)], dv)
                for r in range(NSUB):
                    def qq(qi, _):
                        st = r * SUB + qi * 16
                        sj = sv[pl.ds(st, 16)]
                        dj = dv[pl.ds(st, 16)]
                        t = plsc.load_gather(x_v, [sj])
                        dl = dj - lo
                        m = (dl >= 0) & (dl < QROWS)
                        idx2d[r, pl.ds(qi * 16, 16)] = jnp.where(
                            m, dl * V + t, HDUM)
                        return 0
                    lax.fori_loop(0, SUB // 16, qq, 0)
                    pltpu.sync_copy(ones_v, hist_sh.at[idx2d.at[r]],
                                    add=True)
                return 0

            lax.fori_loop(0, NCHUNK, chunk, 0)
            plsc.subcore_barrier()

            hbase = (c * NPH + q) * HWQ + base
            for i in range(4):
                pltpu.sync_copy(hist_sh.at[pl.ds(base + i * HZQ, HZQ)],
                                bounce)
                pltpu.sync_copy(bounce, out_h.at[pl.ds(hbase + i * HZQ,
                                                       HZQ)])

    f = pl.kernel(
        body,
        out_type=jax.ShapeDtypeStruct((NC * NPH * HWQ,), jnp.float32),
        mesh=_mesh(),
        scratch_types=[
            pltpu.VMEM((N,), jnp.int32),
            pltpu.VMEM((CH,), jnp.int32),
            pltpu.VMEM((CH,), jnp.int32),
            pltpu.VMEM((NSUB, SUB), jnp.int32),
            pltpu.VMEM((SUB,), jnp.float32),
            pltpu.VMEM((HZQ,), jnp.float32),
            pltpu.VMEM_SHARED((HWQ,), jnp.float32),
        ],
        compiler_params=pltpu.CompilerParams(needs_layout_passes=False),
    )
    return f(srcp, dstp, x)


def _sc_rowsum(srcp, dstp, h1a, h1b):
    def body(src_h, dst_h, ha_h, hb_h, out_h, sv, dv, gidx, sidx, r0, r1,
             bounce, sem0, sem1, agg_sh):
        c = lax.axis_index("c")
        s = lax.axis_index("s")
        lo = c * HALF

        zv = jnp.zeros((16,), jnp.float32)
        ebase = s * EPT
        rbase = s * RSPANQ

        for q in range(NPH):
            h_h = ha_h if q == 0 else hb_h

            def zi(i, _):
                bounce[i // 2, pl.ds((i % 2) * 16, 16)] = zv
                return 0

            lax.fori_loop(0, RZQ * (WQ // 16), zi, 0)
            for i in range(4):
                pltpu.sync_copy(bounce,
                                agg_sh.at[pl.ds(rbase + i * RZQ, RZQ)])
            plsc.subcore_barrier()

            def chunk(ci, _):
                off = ebase + ci * CH
                pltpu.sync_copy(src_h.at[pl.ds(off, CH)], sv)
                pltpu.sync_copy(dst_h.at[pl.ds(off, CH)], dv)
                for r in range(NSUB):
                    def qq(qi, _):
                        st = r * SUB + qi * 16
                        sj = sv[pl.ds(st, 16)]
                        dj = dv[pl.ds(st, 16)]
                        dl = dj - lo
                        m = (dl >= 0) & (dl < HALF)
                        gidx[r, pl.ds(qi * 16, 16)] = sj
                        sidx[r, pl.ds(qi * 16, 16)] = jnp.where(m, dl,
                                                                RDUM)
                        return 0
                    lax.fori_loop(0, SUB // 16, qq, 0)
                d_cur = pltpu.async_copy(h_h.at[gidx.at[0]], r0, sem0)
                for r in range(NSUB):
                    if r + 1 < NSUB:
                        nbuf = r1 if r % 2 == 0 else r0
                        nsem = sem1 if r % 2 == 0 else sem0
                        d_next = pltpu.async_copy(h_h.at[gidx.at[r + 1]],
                                                  nbuf, nsem)
                    d_cur.wait()
                    buf = r0 if r % 2 == 0 else r1
                    pltpu.sync_copy(buf, agg_sh.at[sidx.at[r]], add=True)
                    if r + 1 < NSUB:
                        d_cur = d_next
                return 0

            lax.fori_loop(0, NCHUNK, chunk, 0)
            plsc.subcore_barrier()

            obase = (c * NPH + q) * RQR + rbase
            for i in range(4):
                pltpu.sync_copy(agg_sh.at[pl.ds(rbase + i * RZQ, RZQ)],
                                bounce)
                pltpu.sync_copy(bounce, out_h.at[pl.ds(obase + i * RZQ,
                                                       RZQ)])

    f = pl.kernel(
        body,
        out_type=jax.ShapeDtypeStruct((NC * NPH * RQR, WQ), jnp.float32),
        mesh=_mesh(),
        scratch_types=[
            pltpu.VMEM((CH,), jnp.int32),
            pltpu.VMEM((CH,), jnp.int32),
            pltpu.VMEM((NSUB, SUB), jnp.int32),
            pltpu.VMEM((NSUB, SUB), jnp.int32),
            pltpu.VMEM((SUB, WQ), jnp.float32),
            pltpu.VMEM((SUB, WQ), jnp.float32),
            pltpu.VMEM((RZQ, WQ), jnp.float32),
            pltpu.SemaphoreType.DMA,
            pltpu.SemaphoreType.DMA,
            pltpu.VMEM_SHARED((RQR, WQ), jnp.float32),
        ],
        compiler_params=pltpu.CompilerParams(needs_layout_passes=False,
                                             use_tc_tiling_on_sc=False),
    )
    return f(srcp, dstp, h1a, h1b)


def _tc_layer1(counts2d, x3, emb, W1l, W1r, b1):
    def body(c_ref, x_ref, emb_ref, wl_ref, wr_ref, b_ref, oa_ref, ob_ref):
        cb = c_ref[...]
        denom = jnp.maximum(jnp.sum(cb, axis=1, keepdims=True), 1.0)
        Ml = jnp.dot(emb_ref[...], wl_ref[...],
                     preferred_element_type=jnp.float32)
        Mr = jnp.dot(emb_ref[...], wr_ref[...],
                     preferred_element_type=jnp.float32)
        xb = x_ref[0, 0, :]
        oh = (xb[:, None] == lax.broadcasted_iota(jnp.int32, (BN, V), 1)
              ).astype(jnp.float32)
        h1 = (jnp.dot(cb, Ml, preferred_element_type=jnp.float32) / denom
              + jnp.dot(oh, Mr, preferred_element_type=jnp.float32)
              + b_ref[...][None, :])
        h1 = jnp.maximum(h1, 0.0)
        oa_ref[...] = h1[:, :WQ]
        ob_ref[...] = h1[:, WQ:]

    return pl.pallas_call(
        body,
        grid=(NB,),
        in_specs=[
            pl.BlockSpec((BN, V), lambda i: (i, 0)),
            pl.BlockSpec((1, 1, BN), lambda i: (i, 0, 0)),
            pl.BlockSpec((V, D), lambda i: (0, 0)),
            pl.BlockSpec((D, H), lambda i: (0, 0)),
            pl.BlockSpec((D, H), lambda i: (0, 0)),
            pl.BlockSpec((H,), lambda i: (0,)),
        ],
        out_specs=[
            pl.BlockSpec((BN, WQ), lambda i: (i, 0)),
            pl.BlockSpec((BN, WQ), lambda i: (i, 0)),
        ],
        out_shape=[
            jax.ShapeDtypeStruct((N, WQ), jnp.float32),
            jax.ShapeDtypeStruct((N, WQ), jnp.float32),
        ],
    )(counts2d, x3, emb, W1l, W1r, b1)


def _tc_layer2(h1a, h1b, aggA, aggB, counts2d, b3, W2l, W2r, b2, WlinP,
               blinP):
    def body(ha_ref, hb_ref, aa_ref, ab_ref, c_ref, b3_ref, wl_ref,
             wr_ref, b2_ref, wlin_ref, blin_ref, o_ref, sums_sc, cnt_sc):
        i = pl.program_id(0)

        @pl.when(i == 0)
        def _():
            sums_sc[...] = jnp.zeros_like(sums_sc)
            cnt_sc[...] = jnp.zeros_like(cnt_sc)

        cb = c_ref[...]
        denom = jnp.maximum(jnp.sum(cb, axis=1, keepdims=True), 1.0)
        wl = wl_ref[...]
        wr = wr_ref[...]
        h2 = jnp.maximum(
            jnp.dot(aa_ref[...] / denom, wl[:WQ, :],
                    preferred_element_type=jnp.float32)
            + jnp.dot(ab_ref[...] / denom, wl[WQ:, :],
                      preferred_element_type=jnp.float32)
            + jnp.dot(ha_ref[...], wr[:WQ, :],
                      preferred_element_type=jnp.float32)
            + jnp.dot(hb_ref[...], wr[WQ:, :],
                      preferred_element_type=jnp.float32)
            + b2_ref[...][None, :], 0.0)
        bb = b3_ref[0, 0, :]
        oh = (bb[:, None] == lax.broadcasted_iota(jnp.int32, (BN, G), 1)
              ).astype(jnp.float32)
        dn = (((0,), (0,)), ((), ()))
        sums_sc[...] += lax.dot_general(
            oh, h2, dn, preferred_element_type=jnp.float32)
        cnt_sc[...] += lax.dot_general(
            oh, jnp.ones((BN, H), jnp.float32), dn,
            preferred_element_type=jnp.float32)

        @pl.when(i == NB - 1)
        def _():
            pooled = sums_sc[...] / jnp.maximum(cnt_sc[...], 1.0)
            o_ref[...] = (jnp.dot(pooled, wlin_ref[...],
                                  preferred_element_type=jnp.float32)
                          + blin_ref[...][None, :])

    return pl.pallas_call(
        body,
        grid=(NB,),
        in_specs=[
            pl.BlockSpec((BN, WQ), lambda i: (i, 0)),
            pl.BlockSpec((BN, WQ), lambda i: (i, 0)),
            pl.BlockSpec((BN, WQ), lambda i: (i, 0)),
            pl.BlockSpec((BN, WQ), lambda i: (i, 0)),
            pl.BlockSpec((BN, V), lambda i: (i, 0)),
            pl.BlockSpec((1, 1, BN), lambda i: (i, 0, 0)),
            pl.BlockSpec((H, H), lambda i: (0, 0)),
            pl.BlockSpec((H, H), lambda i: (0, 0)),
            pl.BlockSpec((H,), lambda i: (0,)),
            pl.BlockSpec((H, 128), lambda i: (0, 0)),
            pl.BlockSpec((128,), lambda i: (0,)),
        ],
        out_specs=pl.BlockSpec((G, 128), lambda i: (0, 0)),
        out_shape=jax.ShapeDtypeStruct((G, 128), jnp.float32),
        scratch_shapes=[
            pltpu.VMEM((G, H), jnp.float32),
            pltpu.VMEM((G, H), jnp.float32),
        ],
    )(h1a, h1b, aggA, aggB, counts2d, b3, W2l, W2r, b2, WlinP, blinP)


def kernel(x, edge_index, batch, emb, W1l, W1r, b1, W2l, W2r, b2, Wlin, blin):
    x = x.astype(jnp.int32)
    src = edge_index[0].astype(jnp.int32)
    dst = edge_index[1].astype(jnp.int32)
    npad = EPAD - E
    srcp = jnp.concatenate([src, jnp.zeros((npad,), jnp.int32)])
    dstp = jnp.concatenate([dst, jnp.full((npad,), N, jnp.int32)])

    cf = _sc_hist(srcp, dstp, x)
    counts2d = jnp.concatenate(
        [cf[k * HWQ: k * HWQ + QROWS * V] for k in range(NC * NPH)]
    ).reshape(N, V)

    x3 = x.reshape(NB, 1, BN)
    h1a, h1b = _tc_layer1(counts2d, x3, emb, W1l, W1r, b1)

    aggp = _sc_rowsum(srcp, dstp, h1a, h1b)
    aggA = jnp.concatenate([aggp[0 * RQR: 0 * RQR + HALF],
                            aggp[2 * RQR: 2 * RQR + HALF]], axis=0)
    aggB = jnp.concatenate([aggp[1 * RQR: 1 * RQR + HALF],
                            aggp[3 * RQR: 3 * RQR + HALF]], axis=0)

    b3 = batch.astype(jnp.int32).reshape(NB, 1, BN)
    WlinP = jnp.pad(Wlin, ((0, 0), (0, 128 - Wlin.shape[1])))
    blinP = jnp.pad(blin, (0, 128 - blin.shape[0]))
    outp = _tc_layer2(h1a, h1b, aggA, aggB, counts2d, b3, W2l, W2r, b2,
                      WlinP, blinP)
    return outp[:, :2]
